# baseline (speedup 1.0000x reference)
import jax
import jax.numpy as jnp
from jax.experimental import pallas as pl
from jax.experimental.pallas import tpu as pltpu


def _pick_bm(n: int) -> int:
    for bm in (400, 500, 250, 200, 125, 100, 80, 50, 40, 25, 20, 16, 10, 8):
        if n % bm == 0:
            return bm
    return n


def _xw_kernel(x_ref, w_ref, out_ref):
    out_ref[...] = jax.lax.dot_general(
        x_ref[...].astype(jnp.bfloat16),
        w_ref[...].astype(jnp.bfloat16),
        (((1,), (0,)), ((), ())),
        preferred_element_type=jnp.float32,
    ).astype(jnp.bfloat16)


def _phi_scale_kernel(phi_ref, v_ref, k_ref, out_ref):
    acc = jax.lax.dot_general(
        phi_ref[...].astype(jnp.bfloat16),
        v_ref[...],
        (((1,), (0,)), ((), ())),
        preferred_element_type=jnp.float32,
    )
    out_ref[...] = (k_ref[...] * acc).astype(jnp.bfloat16)


def _phi_relu_kernel(phi_ref, v_ref, out_ref):
    acc = jax.lax.dot_general(
        phi_ref[...].astype(jnp.bfloat16),
        v_ref[...],
        (((1,), (0,)), ((), ())),
        preferred_element_type=jnp.float32,
    )
    out_ref[...] = jnp.maximum(acc, 0.0)


def kernel(x, phi0, phi1, W, kernel):
    n, d_in = x.shape
    d_out = W.shape[1]
    bm = _pick_bm(n)
    grid = (n // bm,)

    xp = pl.pallas_call(
        _xw_kernel,
        out_shape=jax.ShapeDtypeStruct((n, d_out), jnp.bfloat16),
    )(x, W)

    t = pl.pallas_call(
        _phi_scale_kernel,
        grid=grid,
        in_specs=[
            pl.BlockSpec((bm, n), lambda i: (i, 0)),
            pl.BlockSpec((n, d_out), lambda i: (0, 0)),
            pl.BlockSpec((bm, 1), lambda i: (i, 0)),
        ],
        out_specs=pl.BlockSpec((bm, d_out), lambda i: (i, 0)),
        out_shape=jax.ShapeDtypeStruct((n, d_out), jnp.bfloat16),
    )(phi0, xp, kernel)

    out = pl.pallas_call(
        _phi_relu_kernel,
        grid=grid,
        in_specs=[
            pl.BlockSpec((bm, n), lambda i: (i, 0)),
            pl.BlockSpec((n, d_out), lambda i: (0, 0)),
        ],
        out_specs=pl.BlockSpec((bm, d_out), lambda i: (i, 0)),
        out_shape=jax.ShapeDtypeStruct((n, d_out), jnp.float32),
    )(phi1, t)

    return out
